# baseline (speedup 1.0000x reference)
import functools
import jax
import jax.numpy as jnp
from jax.experimental import pallas as pl
from jax.experimental.pallas import tpu as pltpu

NS = 5000
NC = 5000
NP = 5120
D = 128
E = 320000
BLK = 128
NBLK = NP // BLK


def _proj_body(sx, cx, es, ec, Ws, bs, Wc, bc, xs, xc):
    xs[...] = jnp.dot(sx[...], Ws[...], preferred_element_type=jnp.float32) + bs[...] + es[...]
    xc[...] = jnp.dot(cx[...], Wc[...], preferred_element_type=jnp.float32) + bc[...] + ec[...]


def _project(student_x, code_x, emb_s, emb_c, Ws, bs, Wc, bc):
    row = pl.BlockSpec((BLK, D), lambda i: (i, 0))
    full = pl.BlockSpec((1, D), lambda i: (0, 0))
    return pl.pallas_call(
        _proj_body,
        grid=(NBLK,),
        in_specs=[row, row, row, row,
                  pl.BlockSpec((D, D), lambda i: (0, 0)), full,
                  pl.BlockSpec((D, D), lambda i: (0, 0)), full],
        out_specs=[row, row],
        out_shape=[jax.ShapeDtypeStruct((NP, D), jnp.float32),
                   jax.ShapeDtypeStruct((NP, D), jnp.float32)],
    )(student_x, code_x, emb_s, emb_c, Ws, bs, Wc, bc)


def _layer1_body(C_ref, xs_ref, xc_ref, rc_ref, rs_ref,
                 Wrtk, Wntk, btk, Wrrv, Wnrv, brv,
                 hc_ref, hs_ref, acc_ref):
    i = pl.program_id(0)
    cblk = C_ref[...]
    xs = xs_ref[...]
    xci = xc_ref[...]
    sums_c = jnp.dot(cblk, xs, preferred_element_type=jnp.float32)
    mean_c = sums_c * rc_ref[...]
    hc = (jnp.dot(xci, Wrtk[...], preferred_element_type=jnp.float32)
          + jnp.dot(mean_c, Wntk[...], preferred_element_type=jnp.float32)
          + btk[...])
    hc_ref[...] = jnp.maximum(hc, 0.0)
    contrib = jax.lax.dot_general(cblk, xci, (((0,), (0,)), ((), ())),
                                  preferred_element_type=jnp.float32)

    @pl.when(i == 0)
    def _():
        acc_ref[...] = contrib

    @pl.when(i > 0)
    def _():
        acc_ref[...] += contrib

    @pl.when(i == NBLK - 1)
    def _():
        mean_s = acc_ref[...] * rs_ref[...]
        hs = (jnp.dot(xs, Wrrv[...], preferred_element_type=jnp.float32)
              + jnp.dot(mean_s, Wnrv[...], preferred_element_type=jnp.float32)
              + brv[...])
        hs_ref[...] = jnp.maximum(hs, 0.0)


def _layer1(C, xs, xc, rc_b, rs_b, Wrtk, Wntk, btk, Wrrv, Wnrv, brv):
    row = pl.BlockSpec((BLK, D), lambda i: (i, 0))
    fullnp = pl.BlockSpec((NP, D), lambda i: (0, 0))
    wspec = pl.BlockSpec((D, D), lambda i: (0, 0))
    bspec = pl.BlockSpec((1, D), lambda i: (0, 0))
    return pl.pallas_call(
        _layer1_body,
        grid=(NBLK,),
        in_specs=[pl.BlockSpec((BLK, NP), lambda i: (i, 0)),
                  fullnp,
                  row,
                  row,
                  fullnp,
                  wspec, wspec, bspec, wspec, wspec, bspec],
        out_specs=[row, fullnp],
        out_shape=[jax.ShapeDtypeStruct((NP, D), jnp.float32),
                   jax.ShapeDtypeStruct((NP, D), jnp.float32)],
        scratch_shapes=[pltpu.VMEM((NP, D), jnp.float32)],
    )(C, xs, xc, rc_b, rs_b, Wrtk, Wntk, btk, Wrrv, Wnrv, brv)


def _layer2_body(C_ref, hs_ref, hc_ref, rc_ref, rs_ref,
                 wrc, wnc, wrs, wns, bc2, bs2,
                 ac_ref, as_ref, vs_ref, accs_ref):
    i = pl.program_id(0)
    cblk = C_ref[...]
    hci = hc_ref[...]

    @pl.when(i == 0)
    def _():
        vs_ref[...] = jnp.dot(hs_ref[...], wnc[...], preferred_element_type=jnp.float32)

    sums = jnp.dot(cblk, vs_ref[...], preferred_element_type=jnp.float32)
    ac = (jnp.dot(hci, wrc[...], preferred_element_type=jnp.float32)
          + sums * rc_ref[...] + bc2[...])
    ac_ref[...] = ac
    vci = jnp.dot(hci, wns[...], preferred_element_type=jnp.float32)
    contrib = jax.lax.dot_general(cblk, vci, (((0,), (0,)), ((), ())),
                                  preferred_element_type=jnp.float32)

    @pl.when(i == 0)
    def _():
        accs_ref[...] = contrib

    @pl.when(i > 0)
    def _():
        accs_ref[...] += contrib

    @pl.when(i == NBLK - 1)
    def _():
        as_ref[...] = (jnp.dot(hs_ref[...], wrs[...], preferred_element_type=jnp.float32)
                       + accs_ref[...] * rs_ref[...] + bs2[...])


def _layer2(C, hs, hc, rc1, rs1, wrc, wnc, wrs, wns, bc2, bs2):
    rowd = pl.BlockSpec((BLK, D), lambda i: (i, 0))
    row1 = pl.BlockSpec((BLK, 1), lambda i: (i, 0))
    fullnp = pl.BlockSpec((NP, D), lambda i: (0, 0))
    full1 = pl.BlockSpec((NP, 1), lambda i: (0, 0))
    vspec = pl.BlockSpec((D, 1), lambda i: (0, 0))
    sspec = pl.BlockSpec((1, 1), lambda i: (0, 0))
    return pl.pallas_call(
        _layer2_body,
        grid=(NBLK,),
        in_specs=[pl.BlockSpec((BLK, NP), lambda i: (i, 0)),
                  fullnp, rowd, row1, full1,
                  vspec, vspec, vspec, vspec, sspec, sspec],
        out_specs=[row1, full1],
        out_shape=[jax.ShapeDtypeStruct((NP, 1), jnp.float32),
                   jax.ShapeDtypeStruct((NP, 1), jnp.float32)],
        scratch_shapes=[pltpu.VMEM((NP, 1), jnp.float32),
                        pltpu.VMEM((NP, 1), jnp.float32)],
    )(C, hs, hc, rc1, rs1, wrc, wnc, wrs, wns, bc2, bs2)


def kernel(student_x, code_x, edge_attr, student_node_id, code_node_id, edge_index, params):
    p = params
    src = edge_index[0]
    dst = edge_index[1]

    def padrows(a):
        return jnp.pad(a, ((0, NP - a.shape[0]), (0, 0)))

    sx = padrows(student_x)
    cx = padrows(code_x)
    es = padrows(p['emb_s'][student_node_id])
    ec = padrows(p['emb_c'][code_node_id])

    C = jnp.zeros((NP, NP), jnp.float32).at[dst, src].add(1.0)
    cnt_c = jnp.zeros((NP,), jnp.float32).at[dst].add(1.0)
    cnt_s = jnp.zeros((NP,), jnp.float32).at[src].add(1.0)
    rc = 1.0 / jnp.maximum(cnt_c, 1.0)
    rs = 1.0 / jnp.maximum(cnt_s, 1.0)
    rc_b = jnp.broadcast_to(rc[:, None], (NP, D))
    rs_b = jnp.broadcast_to(rs[:, None], (NP, D))

    b_slin = p['b_slin'][None, :]
    b_clin = p['b_clin'][None, :]
    b1_tk = p['b1_tk'][None, :]
    b1_rv = p['b1_rv'][None, :]

    w_s = p['W_cls'][:D]
    w_c = p['W_cls'][D:2 * D]
    w_e = p['W_cls'][2 * D:]
    wrc = p['W2_tk_root'] @ w_c
    wnc = p['W2_tk_nbr'] @ w_c
    wrs = p['W2_rv_root'] @ w_s
    wns = p['W2_rv_nbr'] @ w_s
    bc2 = (p['b2_tk'] @ w_c)[None, :]
    bs2 = (p['b2_rv'] @ w_s)[None, :]

    xs, xc = _project(sx, cx, es, ec, p['W_slin'], b_slin, p['W_clin'], b_clin)
    hc, hs = _layer1(C, xs, xc, rc_b, rs_b,
                     p['W1_tk_root'], p['W1_tk_nbr'], b1_tk,
                     p['W1_rv_root'], p['W1_rv_nbr'], b1_rv)
    ac, a_s = _layer2(C, hs, hc, rc[:, None], rs[:, None],
                      wrc, wnc, wrs, wns, bc2, bs2)

    ea = edge_attr @ w_e
    out = a_s[src, 0] + ac[dst, 0] + ea[:, 0] + p['b_cls'][0]
    return out
